# baseline (speedup 1.0000x reference)
import functools

import jax
import jax.numpy as jnp
from jax import lax
from jax.experimental import pallas as pl
from jax.experimental.pallas import tpu as pltpu
from jax.experimental.pallas import tpu_sc as plsc

VOCAB = 100000
EMB = 128
N_ROWS = 4096 * 200
NC, NS = 2, 16
NW = NC * NS
CHUNK = 128
NBUF = 5
DEFER = 2

NCHUNK = 175
SC_ROWS = NW * NCHUNK * CHUNK
NGROUP = NCHUNK // NBUF

TC_ROWS = N_ROWS - SC_ROWS
TC_BLK = 1024
TC_NBLK = TC_ROWS // TC_BLK


def _make_sc_lookup():
  mesh = plsc.VectorSubcoreMesh(core_axis_name="c", subcore_axis_name="s")

  @functools.partial(
      pl.kernel,
      mesh=mesh,
      out_type=jax.ShapeDtypeStruct((SC_ROWS, EMB), jnp.float32),
      scratch_types=[
          pltpu.VMEM((NCHUNK, CHUNK), jnp.int32),
          pltpu.VMEM((NBUF, CHUNK, EMB), jnp.float32),
      ]
      + [pltpu.SemaphoreType.DMA] * (2 * NBUF),
  )
  def lookup(table_hbm, idx_hbm, out_hbm, idx_v, bufs, *sems):
    gsem = sems[:NBUF]
    osem = sems[NBUF:]
    wid = lax.axis_index("s") * NC + lax.axis_index("c")
    base = wid * (NCHUNK * CHUNK)
    pltpu.sync_copy(idx_hbm.at[wid], idx_v)

    def start_gather(j, b):
      pltpu.async_copy(table_hbm.at[idx_v.at[j]], bufs.at[b], gsem[b])

    def wait_gather(b):
      pltpu.make_async_copy(
          table_hbm.at[pl.ds(0, CHUNK)], bufs.at[b], gsem[b]).wait()

    def start_out(j, b):
      pltpu.async_copy(
          bufs.at[b], out_hbm.at[pl.ds(base + j * CHUNK, CHUNK)], osem[b])

    def wait_out(b):
      pltpu.make_async_copy(
          bufs.at[b], out_hbm.at[pl.ds(0, CHUNK)], osem[b]).wait()

    def step(j, b, recycle, restart):
      wait_gather(b)
      start_out(j, b)
      b2 = (b - DEFER) % NBUF
      if recycle:
        wait_out(b2)
      if restart:
        start_gather(j + NBUF - DEFER, b2)

    for b in range(NBUF):
      start_gather(b, b)

    for b in range(NBUF):
      step(b, b, recycle=(b >= DEFER), restart=(b >= DEFER))

    def group(g, carry):
      for b in range(NBUF):
        step(g * NBUF + b, b, recycle=True, restart=True)
      return carry

    lax.fori_loop(1, NGROUP - 1, group, 0)

    for b in range(NBUF):
      j = (NGROUP - 1) * NBUF + b
      step(j, b, recycle=True, restart=(j + NBUF - DEFER < NCHUNK))

    for b in range(NBUF - DEFER, NBUF):
      wait_out(b)

  return lookup


def _tc_body(idx_ref, table_ref, out_ref):
  def row(j, carry):
    r = idx_ref[0, 0, j]
    out_ref[pl.ds(j, 1), :] = table_ref[pl.ds(r, 1), :]
    return carry

  lax.fori_loop(0, TC_BLK, row, 0)


def _tc_lookup(table, idx):
  return pl.pallas_call(
      _tc_body,
      grid=(TC_NBLK,),
      in_specs=[
          pl.BlockSpec((1, 1, TC_BLK), lambda i: (i, 0, 0),
                       memory_space=pltpu.SMEM),
          pl.BlockSpec((VOCAB, EMB), lambda i: (0, 0)),
      ],
      out_specs=pl.BlockSpec((TC_BLK, EMB), lambda i: (i, 0)),
      out_shape=jax.ShapeDtypeStruct((TC_ROWS, EMB), jnp.float32),
  )(idx.reshape(TC_NBLK, 1, TC_BLK), table)


_sc_lookup = _make_sc_lookup()


def kernel(source, emb_lut):
  flat = source.reshape(N_ROWS).astype(jnp.int32)
  sc_idx = flat[:SC_ROWS].reshape(NW, NCHUNK, CHUNK)
  tc_idx = flat[SC_ROWS:]
  sc_out = _sc_lookup(emb_lut, sc_idx)
  tc_out = _tc_lookup(emb_lut, tc_idx)
  out = jnp.concatenate([sc_out, tc_out], axis=0)
  return out.reshape(source.shape[0], source.shape[1], EMB)


# speedup vs baseline: 3.1075x; 3.1075x over previous
import functools

import jax
import jax.numpy as jnp
from jax import lax
from jax.experimental import pallas as pl
from jax.experimental.pallas import tpu as pltpu
from jax.experimental.pallas import tpu_sc as plsc

EMB = 128
N_ROWS = 4096 * 200
NC, NS = 2, 16
NW = NC * NS
ROWS_PER_W = N_ROWS // NW
CHUNK = 128
NCHUNK = ROWS_PER_W // CHUNK
GC = 2
BLK = GC * CHUNK
NBLK = ROWS_PER_W // BLK
NBUF = 3


def _make_lookup():
  mesh = plsc.VectorSubcoreMesh(core_axis_name="c", subcore_axis_name="s")

  @functools.partial(
      pl.kernel,
      mesh=mesh,
      out_type=jax.ShapeDtypeStruct((N_ROWS, EMB), jnp.float32),
      scratch_types=[
          pltpu.VMEM((NCHUNK, CHUNK), jnp.int32),
          pltpu.VMEM((NBUF, BLK, EMB), jnp.float32),
      ]
      + [pltpu.SemaphoreType.DMA] * (2 * NBUF),
  )
  def lookup(table_hbm, idx_hbm, out_hbm, idx_v, bufs, *sems):
    gsem = sems[:NBUF]
    osem = sems[NBUF:]
    wid = lax.axis_index("s") * NC + lax.axis_index("c")
    base = wid * ROWS_PER_W
    pltpu.sync_copy(idx_hbm.at[wid], idx_v)

    def start_gathers(i, s):
      for k in range(GC):
        pltpu.async_copy(
            table_hbm.at[idx_v.at[GC * i + k]],
            bufs.at[s, pl.ds(k * CHUNK, CHUNK)], gsem[s])

    def wait_gathers(s):
      pltpu.make_async_copy(
          table_hbm.at[pl.ds(0, BLK)], bufs.at[s], gsem[s]).wait()

    def start_out(i, s):
      pltpu.async_copy(
          bufs.at[s], out_hbm.at[pl.ds(base + i * BLK, BLK)], osem[s])

    def wait_out(s):
      pltpu.make_async_copy(
          bufs.at[s], out_hbm.at[pl.ds(0, BLK)], osem[s]).wait()

    def step(i, s, recycle, restart):
      wait_gathers(s)
      start_out(i, s)
      s2 = (s - 1) % NBUF
      if recycle:
        wait_out(s2)
      if restart:
        start_gathers(i + NBUF - 1, s2)

    start_gathers(0, 0)
    start_gathers(1, 1)

    step(0, 0, recycle=False, restart=True)
    step(1, 1, recycle=True, restart=True)

    def group(g, carry):
      for q in range(NBUF):
        i = 2 + g * NBUF + q
        step(i, (2 + q) % NBUF, recycle=True, restart=True)
      return carry

    lax.fori_loop(0, (NBLK - 4) // NBUF, group, 0)

    step(NBLK - 2, (NBLK - 2) % NBUF, recycle=True, restart=False)
    step(NBLK - 1, (NBLK - 1) % NBUF, recycle=True, restart=False)

    wait_out((NBLK - 1) % NBUF)

  return lookup


_lookup = _make_lookup()


def kernel(source, emb_lut):
  idx = source.reshape(NW, NCHUNK, CHUNK).astype(jnp.int32)
  out = _lookup(emb_lut, idx)
  return out.reshape(source.shape[0], source.shape[1], EMB)


# speedup vs baseline: 3.1224x; 1.0048x over previous
import functools

import jax
import jax.numpy as jnp
from jax import lax
from jax.experimental import pallas as pl
from jax.experimental.pallas import tpu as pltpu
from jax.experimental.pallas import tpu_sc as plsc

EMB = 128
N_ROWS = 4096 * 200
NC, NS = 2, 16
NW = NC * NS
ROWS_PER_W = N_ROWS // NW
CHUNK = 128
NCHUNK = ROWS_PER_W // CHUNK
NBUF = 5
DEFER = 2
NGROUP = NCHUNK // NBUF


def _make_lookup():
  mesh = plsc.VectorSubcoreMesh(core_axis_name="c", subcore_axis_name="s")

  @functools.partial(
      pl.kernel,
      mesh=mesh,
      out_type=jax.ShapeDtypeStruct((N_ROWS, EMB), jnp.float32),
      scratch_types=[
          pltpu.VMEM((NCHUNK, CHUNK), jnp.int32),
          pltpu.VMEM((NBUF, CHUNK, EMB), jnp.float32),
      ]
      + [pltpu.SemaphoreType.DMA] * (2 * NBUF),
  )
  def lookup(table_hbm, idx_hbm, out_hbm, idx_v, bufs, *sems):
    gsem = sems[:NBUF]
    osem = sems[NBUF:]
    wid = lax.axis_index("s") * NC + lax.axis_index("c")
    base = wid * ROWS_PER_W
    pltpu.sync_copy(idx_hbm.at[wid], idx_v)

    def start_gather(j, b):
      pltpu.async_copy(table_hbm.at[idx_v.at[j]], bufs.at[b], gsem[b])

    def wait_gather(b):
      pltpu.make_async_copy(
          table_hbm.at[pl.ds(0, CHUNK)], bufs.at[b], gsem[b]).wait()

    def start_out(j, b):
      pltpu.async_copy(
          bufs.at[b], out_hbm.at[pl.ds(base + j * CHUNK, CHUNK)], osem[b])

    def wait_out(b):
      pltpu.make_async_copy(
          bufs.at[b], out_hbm.at[pl.ds(0, CHUNK)], osem[b]).wait()

    def step(j, b, recycle, restart):
      wait_gather(b)
      start_out(j, b)
      b2 = (b - DEFER) % NBUF
      if recycle:
        wait_out(b2)
      if restart:
        start_gather(j + NBUF - DEFER, b2)

    for b in range(NBUF):
      start_gather(b, b)

    for b in range(NBUF):
      step(b, b, recycle=(b >= DEFER), restart=(b >= DEFER))

    def group(g, carry):
      for b in range(NBUF):
        step(g * NBUF + b, b, recycle=True, restart=True)
      return carry

    lax.fori_loop(1, NGROUP - 1, group, 0)

    for b in range(NBUF):
      j = (NGROUP - 1) * NBUF + b
      step(j, b, recycle=True, restart=(j + NBUF - DEFER < NCHUNK))

    for b in range(NBUF - DEFER, NBUF):
      wait_out(b)

  return lookup


_lookup = _make_lookup()


def kernel(source, emb_lut):
  idx = source.reshape(NW, NCHUNK, CHUNK).astype(jnp.int32)
  out = _lookup(emb_lut, idx)
  return out.reshape(source.shape[0], source.shape[1], EMB)
